# baseline (speedup 1.0000x reference)
import functools
import jax
import jax.numpy as jnp
from jax.experimental import pallas as pl
from jax.experimental.pallas import tpu as pltpu


def _k1_body(h_ref, wrel_ref, andw_ref, andb_ref, ct_ref, *, nand):
    x = h_ref[...]
    for r in range(nand):
        hr = jax.lax.dot_general(x, wrel_ref[r], (((1,), (1,)), ((), ())))
        a = jax.lax.dot_general(hr, andw_ref[...], (((1,), (1,)), ((), ())))
        ct_ref[r] = hr * jax.nn.sigmoid(a + andb_ref[...])


def _k2_body(h_ref, s_ref, t_ref, r_ref, raw_ref, cnt_ref, *, chunk, e, nmean, rbase):
    @pl.when(pl.program_id(0) == 0)
    def _():
        raw_ref[...] = jnp.zeros_like(raw_ref)
        cnt_ref[...] = jnp.zeros_like(cnt_ref)

    base = pl.program_id(0) * chunk

    def body(i, carry):
        s = s_ref[0, 0, i]
        t = t_ref[0, 0, i]
        r = r_ref[0, 0, i]
        valid = base + i < e
        seg = t * 8 + r
        row = seg // 128
        col = seg % 128
        onehot = (jax.lax.broadcasted_iota(jnp.int32, (1, 128), 1) == col)

        @pl.when(valid)
        def _():
            cnt_ref[pl.ds(row, 1), :] += onehot.astype(jnp.float32)

        @pl.when(valid & (r >= rbase))
        def _():
            seg5 = t * nmean + (r - rbase)
            raw_ref[pl.ds(seg5, 1), :] += h_ref[pl.ds(s, 1), :]

        return carry

    jax.lax.fori_loop(0, chunk, body, 0)


def _k3_body(ct_ref, s_ref, t_ref, r_ref, w_ref, macc_ref, *, chunk, e, n, nand, big):
    @pl.when(pl.program_id(0) == 0)
    def _():
        macc_ref[...] = jnp.full_like(macc_ref, big)

    base = pl.program_id(0) * chunk

    def body(i, carry):
        s = s_ref[0, 0, i]
        t = t_ref[0, 0, i]
        r = r_ref[0, 0, i]
        w = w_ref[0, 0, i]
        valid = (base + i < e) & (r < nand)

        @pl.when(valid)
        def _():
            val = ct_ref[pl.ds(r * n + s, 1), :] * w
            mseg = t * nand + r
            macc_ref[pl.ds(mseg, 1), :] = jnp.minimum(macc_ref[pl.ds(mseg, 1), :], val)

        return carry

    jax.lax.fori_loop(0, chunk, body, 0)


def _k4_body(h_ref, raw_ref, cnt_ref, macc_ref, sf_ref, wself_ref, wrel_ref,
             wp_ref, wc_ref, istw_ref, istb_ref, lng_ref, lnb_ref, out_ref,
             *, nrel, nand, dout):
    x = h_ref[...]
    acc = jax.lax.dot_general(x, wself_ref[...], (((1,), (1,)), ((), ())))
    cnt = cnt_ref[...]
    pos = cnt > 0.0
    denom = jnp.maximum(cnt, 1.0)
    for r in range(nand):
        acc = acc + jnp.where(pos[:, r:r + 1], macc_ref[:, r, :], 0.0)
    m_mat = jax.lax.dot_general(wp_ref[...], wc_ref[...], (((0,), (0,)), ((), ())))
    sitg = jax.nn.sigmoid(
        jax.lax.dot_general(sf_ref[...], istw_ref[...], (((1,), (1,)), ((), ())))
        + istb_ref[...])
    inv_sqrt_d = 1.0 / (float(dout) ** 0.5)
    for r in range(nand, nrel):
        mean_r = jax.lax.dot_general(raw_ref[:, r - nand, :], wrel_ref[r],
                                     (((1,), (1,)), ((), ()))) / denom[:, r:r + 1]
        if r in (3, 4):
            th = jax.lax.dot_general(x, wrel_ref[r], (((1,), (1,)), ((), ())))
            attn = jnp.sum(jnp.dot(mean_r, m_mat) * th, axis=-1, keepdims=True)
            g = jax.nn.sigmoid(attn * inv_sqrt_d)
            msg = th * g + mean_r * (1.0 - g)
        elif r == 5:
            msg = mean_r * sitg
        else:
            msg = mean_r
        acc = acc + jnp.where(pos[:, r:r + 1], msg, 0.0)
    o = jnp.maximum(acc, 0.0)
    mu = jnp.mean(o, axis=-1, keepdims=True)
    var = jnp.mean((o - mu) ** 2, axis=-1, keepdims=True)
    out_ref[...] = (o - mu) * jax.lax.rsqrt(var + 1e-5) * lng_ref[...] + lnb_ref[...]


def kernel(h, edge_index, edge_types, edge_weights, situation_features,
           W_self, W_rel, and_W, and_b, if_Wp, if_Wc, ist_W, ist_b, ln_g, ln_b):
    n, din = h.shape
    e = edge_types.shape[0]
    r = W_rel.shape[0]
    dout = W_self.shape[0]
    sit = situation_features.shape[1]
    nand = 3
    nmean = r - nand
    blk = 1000 if n % 1000 == 0 else n
    nblk = n // blk

    f32 = jnp.float32
    i32 = jnp.int32

    ct = pl.pallas_call(
        functools.partial(_k1_body, nand=nand),
        grid=(nblk,),
        in_specs=[
            pl.BlockSpec((blk, din), lambda i: (i, 0)),
            pl.BlockSpec((r, dout, din), lambda i: (0, 0, 0)),
            pl.BlockSpec((dout, dout), lambda i: (0, 0)),
            pl.BlockSpec((1, dout), lambda i: (0, 0)),
        ],
        out_specs=pl.BlockSpec((nand, blk, dout), lambda i: (0, i, 0)),
        out_shape=jax.ShapeDtypeStruct((nand, n, dout), f32),
    )(h, W_rel, and_W, and_b.reshape(1, dout))

    chunk = 5000 if e % 5000 == 0 else e
    nch = -(-e // chunk)
    pad = nch * chunk - e
    src = jnp.pad(edge_index[0].astype(i32), (0, pad)).reshape(nch, 1, chunk)
    tgt = jnp.pad(edge_index[1].astype(i32), (0, pad)).reshape(nch, 1, chunk)
    rel = jnp.pad(edge_types.astype(i32), (0, pad)).reshape(nch, 1, chunk)
    wgt = jnp.pad(edge_weights.astype(f32), (0, pad)).reshape(nch, 1, chunk)

    smem_idx = pl.BlockSpec((1, 1, chunk), lambda i: (i, 0, 0),
                            memory_space=pltpu.SMEM)

    rawsum, cnt = pl.pallas_call(
        functools.partial(_k2_body, chunk=chunk, e=e, nmean=nmean, rbase=nand),
        grid=(nch,),
        in_specs=[
            pl.BlockSpec((n, din), lambda i: (0, 0)),
            smem_idx, smem_idx, smem_idx,
        ],
        out_specs=[
            pl.BlockSpec((n * nmean, din), lambda i: (0, 0)),
            pl.BlockSpec((n * r // 128, 128), lambda i: (0, 0)),
        ],
        out_shape=[
            jax.ShapeDtypeStruct((n * nmean, din), f32),
            jax.ShapeDtypeStruct((n * r // 128, 128), f32),
        ],
    )(h, src, tgt, rel)

    big = 3.0e38
    macc = pl.pallas_call(
        functools.partial(_k3_body, chunk=chunk, e=e, n=n, nand=nand, big=big),
        grid=(nch,),
        in_specs=[
            pl.BlockSpec((nand * n, dout), lambda i: (0, 0)),
            smem_idx, smem_idx, smem_idx, smem_idx,
        ],
        out_specs=pl.BlockSpec((nand * n, dout), lambda i: (0, 0)),
        out_shape=jax.ShapeDtypeStruct((nand * n, dout), f32),
    )(ct.reshape(nand * n, dout), src, tgt, rel, wgt)

    out = pl.pallas_call(
        functools.partial(_k4_body, nrel=r, nand=nand, dout=dout),
        grid=(nblk,),
        in_specs=[
            pl.BlockSpec((blk, din), lambda i: (i, 0)),
            pl.BlockSpec((blk, nmean, din), lambda i: (i, 0, 0)),
            pl.BlockSpec((blk, r), lambda i: (i, 0)),
            pl.BlockSpec((blk, nand, dout), lambda i: (i, 0, 0)),
            pl.BlockSpec((blk, sit), lambda i: (i, 0)),
            pl.BlockSpec((dout, din), lambda i: (0, 0)),
            pl.BlockSpec((r, dout, din), lambda i: (0, 0, 0)),
            pl.BlockSpec((dout, dout), lambda i: (0, 0)),
            pl.BlockSpec((dout, dout), lambda i: (0, 0)),
            pl.BlockSpec((dout, sit), lambda i: (0, 0)),
            pl.BlockSpec((1, dout), lambda i: (0, 0)),
            pl.BlockSpec((1, dout), lambda i: (0, 0)),
            pl.BlockSpec((1, dout), lambda i: (0, 0)),
        ],
        out_specs=pl.BlockSpec((blk, dout), lambda i: (i, 0)),
        out_shape=jax.ShapeDtypeStruct((n, dout), f32),
    )(h, rawsum.reshape(n, nmean, din), cnt.reshape(n, r),
      macc.reshape(n, nand, dout), situation_features, W_self, W_rel,
      if_Wp, if_Wc, ist_W, ist_b.reshape(1, dout), ln_g.reshape(1, dout),
      ln_b.reshape(1, dout))
    return out


# speedup vs baseline: 2.0822x; 2.0822x over previous
import functools
import jax
import jax.numpy as jnp
from jax import lax
from jax.experimental import pallas as pl
from jax.experimental.pallas import tpu as pltpu
from jax.experimental.pallas import tpu_sc as plsc


def _sc_edge_body(n, krows, ht_hbm, src_hbm, tgt_hbm, rel_hbm, zeros_hbm,
                  sum_hbm, cnt_hbm, shared, srcv, tgtv, relv, sidx, aux,
                  rows, ones_v, sem):
    c = lax.axis_index("c")
    s = lax.axis_index("s")
    wid = s * 2 + c
    nsum = n * 5
    sum_zone = nsum + 16
    sum_dump = nsum + 8
    nhalf = n * 4
    cnt_dump = nhalf

    pltpu.sync_copy(src_hbm.at[wid], srcv)
    pltpu.sync_copy(tgt_hbm.at[wid], tgtv)
    pltpu.sync_copy(rel_hbm.at[wid], relv)

    def jbody(j, carry):
        def lbody(l, cc):
            t = tgtv[j, pl.ds(l * 16, 16)]
            r = relv[j, pl.ds(l * 16, 16)]
            sidx[j, pl.ds(l * 16, 16)] = jnp.where(
                (r >= 3) & (r < 8), t * 5 + (r - 3), sum_dump)
            return cc
        return lax.fori_loop(0, 8, lbody, carry)
    lax.fori_loop(0, krows, jbody, 0)

    for p in range(8):
        @pl.when(s == 0)
        def _():
            pltpu.sync_copy(zeros_hbm.at[pl.ds(0, sum_zone)],
                            shared.at[pl.ds(0, sum_zone)])
        plsc.subcore_barrier()

        def fill(j, carry):
            def lb(l, cc):
                aux[j, pl.ds(l * 16, 16)] = srcv[j, pl.ds(l * 16, 16)] + p * n
                return cc
            return lax.fori_loop(0, 8, lb, carry)
        lax.fori_loop(0, krows, fill, 0)

        def stream(j, carry):
            pltpu.async_copy(ht_hbm.at[aux.at[j]], rows, sem).wait()
            pltpu.sync_copy(rows, shared.at[sidx.at[j]], add=True)
            return carry
        lax.fori_loop(0, krows, stream, 0)

        plsc.subcore_barrier()

        @pl.when(s == 0)
        def _():
            pltpu.sync_copy(shared.at[pl.ds(0, nsum)], sum_hbm.at[c, p])
        plsc.subcore_barrier()

    def ob(j, carry):
        ones_v[j, :] = jnp.full((16,), 1.0, jnp.float32)
        return carry
    lax.fori_loop(0, 128, ob, 0)

    for ch in range(2):
        @pl.when(s == 0)
        def _():
            pltpu.sync_copy(zeros_hbm.at[pl.ds(0, nhalf + 8)],
                            shared.at[pl.ds(0, nhalf + 8)])
        plsc.subcore_barrier()

        def cfill(j, carry):
            def lb(l, cc):
                t = tgtv[j, pl.ds(l * 16, 16)]
                r = relv[j, pl.ds(l * 16, 16)]
                ci = t * 8 + r - ch * nhalf
                ok = (r < 8) & (ci >= 0) & (ci < nhalf)
                aux[j, pl.ds(l * 16, 16)] = jnp.where(ok, ci, cnt_dump)
                return cc
            return lax.fori_loop(0, 8, lb, carry)
        lax.fori_loop(0, krows, cfill, 0)

        def streamb(j, carry):
            pltpu.sync_copy(ones_v, shared.at[aux.at[j]], add=True)
            return carry
        lax.fori_loop(0, krows, streamb, 0)

        plsc.subcore_barrier()

        @pl.when(s == 0)
        def _():
            pltpu.sync_copy(shared.at[pl.ds(0, nhalf)], cnt_hbm.at[c, ch])
        plsc.subcore_barrier()


def _sc_edge_sums(h, src, tgt, rel, n, e):
    f32, i32 = jnp.float32, jnp.int32
    e2 = -(-e // 4096) * 4096
    slab = e2 // 32
    krows = slab // 128
    pad = e2 - e
    src2 = jnp.pad(src, (0, pad)).reshape(32, krows, 128)
    tgt2 = jnp.pad(tgt, (0, pad)).reshape(32, krows, 128)
    rel2 = jnp.pad(rel, (0, pad), constant_values=1000).reshape(32, krows, 128)
    ht = h.reshape(n, 8, 16).transpose(1, 0, 2).reshape(8 * n, 16)
    zrows = n * 5 + 16
    zeros = jnp.zeros((zrows, 16), f32)

    kern = functools.partial(
        pl.kernel,
        mesh=plsc.VectorSubcoreMesh(core_axis_name="c", subcore_axis_name="s"),
        compiler_params=pltpu.CompilerParams(use_tc_tiling_on_sc=False),
        out_type=[
            jax.ShapeDtypeStruct((2, 8, n * 5, 16), f32),
            jax.ShapeDtypeStruct((2, 2, n * 4, 16), f32),
        ],
        scratch_types=[
            pltpu.VMEM_SHARED((zrows, 16), f32),
            pltpu.VMEM((krows, 128), i32),
            pltpu.VMEM((krows, 128), i32),
            pltpu.VMEM((krows, 128), i32),
            pltpu.VMEM((krows, 128), i32),
            pltpu.VMEM((krows, 128), i32),
            pltpu.VMEM((128, 16), f32),
            pltpu.VMEM((128, 16), f32),
            pltpu.SemaphoreType.DMA,
        ],
    )(functools.partial(_sc_edge_body, n, krows))
    sums, cnt = kern(ht, src2, tgt2, rel2, zeros)
    raw = sums.transpose(0, 2, 1, 3).reshape(2, n, 5, 128)
    cnt2 = cnt[:, :, :, 0].reshape(2, n, 8)
    return raw, cnt2


def _k1_body(h_ref, wrel_ref, andw_ref, andb_ref, ct_ref, *, nand):
    x = h_ref[...]
    for r in range(nand):
        hr = jax.lax.dot_general(x, wrel_ref[r], (((1,), (1,)), ((), ())))
        a = jax.lax.dot_general(hr, andw_ref[...], (((1,), (1,)), ((), ())))
        ct_ref[r] = hr * jax.nn.sigmoid(a + andb_ref[...])


def _k2_body(h_ref, s_ref, t_ref, r_ref, raw_ref, cnt_ref, *, chunk, e, nmean, rbase):
    @pl.when(pl.program_id(0) == 0)
    def _():
        raw_ref[...] = jnp.zeros_like(raw_ref)
        cnt_ref[...] = jnp.zeros_like(cnt_ref)

    base = pl.program_id(0) * chunk

    def body(i, carry):
        s = s_ref[0, 0, i]
        t = t_ref[0, 0, i]
        r = r_ref[0, 0, i]
        valid = base + i < e
        seg = t * 8 + r
        row = seg // 128
        col = seg % 128
        onehot = (jax.lax.broadcasted_iota(jnp.int32, (1, 128), 1) == col)

        @pl.when(valid)
        def _():
            cnt_ref[pl.ds(row, 1), :] += onehot.astype(jnp.float32)

        @pl.when(valid & (r >= rbase))
        def _():
            seg5 = t * nmean + (r - rbase)
            raw_ref[pl.ds(seg5, 1), :] += h_ref[pl.ds(s, 1), :]

        return carry

    jax.lax.fori_loop(0, chunk, body, 0)


def _k3_body(ct_ref, s_ref, t_ref, r_ref, w_ref, macc_ref, *, chunk, e, n, nand, big):
    @pl.when(pl.program_id(0) == 0)
    def _():
        macc_ref[...] = jnp.full_like(macc_ref, big)

    base = pl.program_id(0) * chunk

    def body(i, carry):
        s = s_ref[0, 0, i]
        t = t_ref[0, 0, i]
        r = r_ref[0, 0, i]
        w = w_ref[0, 0, i]
        valid = (base + i < e) & (r < nand)

        @pl.when(valid)
        def _():
            val = ct_ref[pl.ds(r * n + s, 1), :] * w
            mseg = t * nand + r
            macc_ref[pl.ds(mseg, 1), :] = jnp.minimum(macc_ref[pl.ds(mseg, 1), :], val)

        return carry

    jax.lax.fori_loop(0, chunk, body, 0)


def _k4_body(h_ref, raw_ref, rawb_ref, cnt_ref, cntb_ref, macc_ref, sf_ref,
             wself_ref, wrel_ref, wp_ref, wc_ref, istw_ref, istb_ref,
             lng_ref, lnb_ref, out_ref, *, nrel, nand, dout):
    x = h_ref[...]
    acc = jax.lax.dot_general(x, wself_ref[...], (((1,), (1,)), ((), ())))
    cnt = cnt_ref[...] + cntb_ref[...]
    pos = cnt > 0.0
    denom = jnp.maximum(cnt, 1.0)
    for r in range(nand):
        acc = acc + jnp.where(pos[:, r:r + 1], macc_ref[:, r, :], 0.0)
    m_mat = jax.lax.dot_general(wp_ref[...], wc_ref[...], (((0,), (0,)), ((), ())))
    sitg = jax.nn.sigmoid(
        jax.lax.dot_general(sf_ref[...], istw_ref[...], (((1,), (1,)), ((), ())))
        + istb_ref[...])
    inv_sqrt_d = 1.0 / (float(dout) ** 0.5)
    for r in range(nand, nrel):
        raw_r = raw_ref[:, r - nand, :] + rawb_ref[:, r - nand, :]
        mean_r = jax.lax.dot_general(raw_r, wrel_ref[r],
                                     (((1,), (1,)), ((), ()))) / denom[:, r:r + 1]
        if r in (3, 4):
            th = jax.lax.dot_general(x, wrel_ref[r], (((1,), (1,)), ((), ())))
            attn = jnp.sum(jnp.dot(mean_r, m_mat) * th, axis=-1, keepdims=True)
            g = jax.nn.sigmoid(attn * inv_sqrt_d)
            msg = th * g + mean_r * (1.0 - g)
        elif r == 5:
            msg = mean_r * sitg
        else:
            msg = mean_r
        acc = acc + jnp.where(pos[:, r:r + 1], msg, 0.0)
    o = jnp.maximum(acc, 0.0)
    mu = jnp.mean(o, axis=-1, keepdims=True)
    var = jnp.mean((o - mu) ** 2, axis=-1, keepdims=True)
    out_ref[...] = (o - mu) * jax.lax.rsqrt(var + 1e-5) * lng_ref[...] + lnb_ref[...]


def kernel(h, edge_index, edge_types, edge_weights, situation_features,
           W_self, W_rel, and_W, and_b, if_Wp, if_Wc, ist_W, ist_b, ln_g, ln_b):
    n, din = h.shape
    e = edge_types.shape[0]
    r = W_rel.shape[0]
    dout = W_self.shape[0]
    sit = situation_features.shape[1]
    nand = 3
    nmean = r - nand
    blk = 1000 if n % 1000 == 0 else n
    nblk = n // blk

    f32 = jnp.float32
    i32 = jnp.int32

    ct = pl.pallas_call(
        functools.partial(_k1_body, nand=nand),
        grid=(nblk,),
        in_specs=[
            pl.BlockSpec((blk, din), lambda i: (i, 0)),
            pl.BlockSpec((r, dout, din), lambda i: (0, 0, 0)),
            pl.BlockSpec((dout, dout), lambda i: (0, 0)),
            pl.BlockSpec((1, dout), lambda i: (0, 0)),
        ],
        out_specs=pl.BlockSpec((nand, blk, dout), lambda i: (0, i, 0)),
        out_shape=jax.ShapeDtypeStruct((nand, n, dout), f32),
    )(h, W_rel, and_W, and_b.reshape(1, dout))

    chunk = 5000 if e % 5000 == 0 else e
    nch = -(-e // chunk)
    pad = nch * chunk - e
    src = jnp.pad(edge_index[0].astype(i32), (0, pad)).reshape(nch, 1, chunk)
    tgt = jnp.pad(edge_index[1].astype(i32), (0, pad)).reshape(nch, 1, chunk)
    rel = jnp.pad(edge_types.astype(i32), (0, pad)).reshape(nch, 1, chunk)
    wgt = jnp.pad(edge_weights.astype(f32), (0, pad)).reshape(nch, 1, chunk)

    smem_idx = pl.BlockSpec((1, 1, chunk), lambda i: (i, 0, 0),
                            memory_space=pltpu.SMEM)

    if e >= 4096:
        raw2, cnt2 = _sc_edge_sums(h, edge_index[0].astype(i32),
                                   edge_index[1].astype(i32),
                                   edge_types.astype(i32), n, e)
        raw_a, raw_b = raw2[0], raw2[1]
        cnt_a, cnt_b = cnt2[0], cnt2[1]
    else:
        rawsum, cnt = pl.pallas_call(
            functools.partial(_k2_body, chunk=chunk, e=e, nmean=nmean,
                              rbase=nand),
            grid=(nch,),
            in_specs=[
                pl.BlockSpec((n, din), lambda i: (0, 0)),
                smem_idx, smem_idx, smem_idx,
            ],
            out_specs=[
                pl.BlockSpec((n * nmean, din), lambda i: (0, 0)),
                pl.BlockSpec((n * r // 128, 128), lambda i: (0, 0)),
            ],
            out_shape=[
                jax.ShapeDtypeStruct((n * nmean, din), f32),
                jax.ShapeDtypeStruct((n * r // 128, 128), f32),
            ],
        )(h, src, tgt, rel)
        raw_a = rawsum.reshape(n, nmean, din)
        raw_b = jnp.zeros_like(raw_a)
        cnt_a = cnt.reshape(n, r)
        cnt_b = jnp.zeros_like(cnt_a)

    big = 3.0e38
    macc = pl.pallas_call(
        functools.partial(_k3_body, chunk=chunk, e=e, n=n, nand=nand, big=big),
        grid=(nch,),
        in_specs=[
            pl.BlockSpec((nand * n, dout), lambda i: (0, 0)),
            smem_idx, smem_idx, smem_idx, smem_idx,
        ],
        out_specs=pl.BlockSpec((nand * n, dout), lambda i: (0, 0)),
        out_shape=jax.ShapeDtypeStruct((nand * n, dout), f32),
    )(ct.reshape(nand * n, dout), src, tgt, rel, wgt)

    out = pl.pallas_call(
        functools.partial(_k4_body, nrel=r, nand=nand, dout=dout),
        grid=(nblk,),
        in_specs=[
            pl.BlockSpec((blk, din), lambda i: (i, 0)),
            pl.BlockSpec((blk, nmean, din), lambda i: (i, 0, 0)),
            pl.BlockSpec((blk, nmean, din), lambda i: (i, 0, 0)),
            pl.BlockSpec((blk, r), lambda i: (i, 0)),
            pl.BlockSpec((blk, r), lambda i: (i, 0)),
            pl.BlockSpec((blk, nand, dout), lambda i: (i, 0, 0)),
            pl.BlockSpec((blk, sit), lambda i: (i, 0)),
            pl.BlockSpec((dout, din), lambda i: (0, 0)),
            pl.BlockSpec((r, dout, din), lambda i: (0, 0, 0)),
            pl.BlockSpec((dout, dout), lambda i: (0, 0)),
            pl.BlockSpec((dout, dout), lambda i: (0, 0)),
            pl.BlockSpec((dout, sit), lambda i: (0, 0)),
            pl.BlockSpec((1, dout), lambda i: (0, 0)),
            pl.BlockSpec((1, dout), lambda i: (0, 0)),
            pl.BlockSpec((1, dout), lambda i: (0, 0)),
        ],
        out_specs=pl.BlockSpec((blk, dout), lambda i: (i, 0)),
        out_shape=jax.ShapeDtypeStruct((n, dout), f32),
    )(h, raw_a, raw_b, cnt_a, cnt_b,
      macc.reshape(n, nand, dout), situation_features, W_self, W_rel,
      if_Wp, if_Wc, ist_W, ist_b.reshape(1, dout), ln_g.reshape(1, dout),
      ln_b.reshape(1, dout))
    return out


# speedup vs baseline: 2.1044x; 1.0106x over previous
import functools
import jax
import jax.numpy as jnp
from jax import lax
from jax.experimental import pallas as pl
from jax.experimental.pallas import tpu as pltpu
from jax.experimental.pallas import tpu_sc as plsc


def _sc_edge_body(n, krows, ht_hbm, src_hbm, tgt_hbm, rel_hbm, zeros_hbm,
                  sum_hbm, cnt_hbm, shared, srcv, tgtv, relv, sidx, aux,
                  rows, ones_v, sem):
    c = lax.axis_index("c")
    s = lax.axis_index("s")
    wid = s * 2 + c
    nsum = n * 5
    sum_zone = nsum + 16
    sum_dump = nsum + 8
    nhalf = n * 4
    cnt_dump = nhalf

    pltpu.sync_copy(src_hbm.at[wid], srcv)
    pltpu.sync_copy(tgt_hbm.at[wid], tgtv)
    pltpu.sync_copy(rel_hbm.at[wid], relv)

    def jbody(j, carry):
        def lbody(l, cc):
            t = tgtv[j, pl.ds(l * 16, 16)]
            r = relv[j, pl.ds(l * 16, 16)]
            sidx[j, pl.ds(l * 16, 16)] = jnp.where(
                (r >= 3) & (r < 8), t * 5 + (r - 3), sum_dump)
            return cc
        return lax.fori_loop(0, 8, lbody, carry)
    lax.fori_loop(0, krows, jbody, 0)

    for p in range(8):
        @pl.when(s == 0)
        def _():
            pltpu.sync_copy(zeros_hbm.at[pl.ds(0, sum_zone)],
                            shared.at[pl.ds(0, sum_zone)])
        plsc.subcore_barrier()

        def fill(j, carry):
            def lb(l, cc):
                aux[j, pl.ds(l * 16, 16)] = srcv[j, pl.ds(l * 16, 16)] + p * n
                return cc
            return lax.fori_loop(0, 8, lb, carry)
        lax.fori_loop(0, krows, fill, 0)

        def stream(j, carry):
            pltpu.async_copy(ht_hbm.at[aux.at[j]], rows, sem).wait()
            pltpu.sync_copy(rows, shared.at[sidx.at[j]], add=True)
            return carry
        lax.fori_loop(0, krows, stream, 0)

        plsc.subcore_barrier()

        @pl.when(s == 0)
        def _():
            pltpu.sync_copy(shared.at[pl.ds(0, nsum)], sum_hbm.at[c, p])
        plsc.subcore_barrier()

    def ob(j, carry):
        ones_v[j, :] = jnp.full((16,), 1.0, jnp.float32)
        return carry
    lax.fori_loop(0, 128, ob, 0)

    for ch in range(2):
        @pl.when(s == 0)
        def _():
            pltpu.sync_copy(zeros_hbm.at[pl.ds(0, nhalf + 8)],
                            shared.at[pl.ds(0, nhalf + 8)])
        plsc.subcore_barrier()

        def cfill(j, carry):
            def lb(l, cc):
                t = tgtv[j, pl.ds(l * 16, 16)]
                r = relv[j, pl.ds(l * 16, 16)]
                ci = t * 8 + r - ch * nhalf
                ok = (r < 8) & (ci >= 0) & (ci < nhalf)
                aux[j, pl.ds(l * 16, 16)] = jnp.where(ok, ci, cnt_dump)
                return cc
            return lax.fori_loop(0, 8, lb, carry)
        lax.fori_loop(0, krows, cfill, 0)

        def streamb(j, carry):
            pltpu.sync_copy(ones_v, shared.at[aux.at[j]], add=True)
            return carry
        lax.fori_loop(0, krows, streamb, 0)

        plsc.subcore_barrier()

        @pl.when(s == 0)
        def _():
            pltpu.sync_copy(shared.at[pl.ds(0, nhalf)], cnt_hbm.at[c, ch])
        plsc.subcore_barrier()


def _sc_edge_sums(h, src, tgt, rel, n, e):
    f32, i32 = jnp.float32, jnp.int32
    e2 = -(-e // 4096) * 4096
    slab = e2 // 32
    krows = slab // 128
    pad = e2 - e
    src2 = jnp.pad(src, (0, pad)).reshape(32, krows, 128)
    tgt2 = jnp.pad(tgt, (0, pad)).reshape(32, krows, 128)
    rel2 = jnp.pad(rel, (0, pad), constant_values=1000).reshape(32, krows, 128)
    ht = h.reshape(n, 8, 16).transpose(1, 0, 2).reshape(8 * n, 16)
    zrows = n * 5 + 16
    zeros = jnp.zeros((zrows, 16), f32)

    kern = functools.partial(
        pl.kernel,
        mesh=plsc.VectorSubcoreMesh(core_axis_name="c", subcore_axis_name="s"),
        compiler_params=pltpu.CompilerParams(use_tc_tiling_on_sc=False),
        out_type=[
            jax.ShapeDtypeStruct((2, 8, n * 5, 16), f32),
            jax.ShapeDtypeStruct((2, 2, n * 4, 16), f32),
        ],
        scratch_types=[
            pltpu.VMEM_SHARED((zrows, 16), f32),
            pltpu.VMEM((krows, 128), i32),
            pltpu.VMEM((krows, 128), i32),
            pltpu.VMEM((krows, 128), i32),
            pltpu.VMEM((krows, 128), i32),
            pltpu.VMEM((krows, 128), i32),
            pltpu.VMEM((128, 16), f32),
            pltpu.VMEM((128, 16), f32),
            pltpu.SemaphoreType.DMA,
        ],
    )(functools.partial(_sc_edge_body, n, krows))
    sums, cnt = kern(ht, src2, tgt2, rel2, zeros)
    raw = sums.transpose(0, 2, 1, 3).reshape(2, n, 5, 128)
    cnt2 = cnt[:, :, :, 0].reshape(2, n, 8)
    return raw, cnt2


def _k1_body(h_ref, wrel_ref, andw_ref, andb_ref, ct_ref, *, nand):
    x = h_ref[...]
    for r in range(nand):
        hr = jax.lax.dot_general(x, wrel_ref[r], (((1,), (1,)), ((), ())))
        a = jax.lax.dot_general(hr, andw_ref[...], (((1,), (1,)), ((), ())))
        ct_ref[r] = hr * jax.nn.sigmoid(a + andb_ref[...])


def _k2_body(h_ref, s_ref, t_ref, r_ref, raw_ref, cnt_ref, *, chunk, e, nmean, rbase):
    @pl.when(pl.program_id(0) == 0)
    def _():
        raw_ref[...] = jnp.zeros_like(raw_ref)
        cnt_ref[...] = jnp.zeros_like(cnt_ref)

    base = pl.program_id(0) * chunk

    def body(i, carry):
        s = s_ref[0, 0, i]
        t = t_ref[0, 0, i]
        r = r_ref[0, 0, i]
        valid = base + i < e
        seg = t * 8 + r
        row = seg // 128
        col = seg % 128
        onehot = (jax.lax.broadcasted_iota(jnp.int32, (1, 128), 1) == col)

        @pl.when(valid)
        def _():
            cnt_ref[pl.ds(row, 1), :] += onehot.astype(jnp.float32)

        @pl.when(valid & (r >= rbase))
        def _():
            seg5 = t * nmean + (r - rbase)
            raw_ref[pl.ds(seg5, 1), :] += h_ref[pl.ds(s, 1), :]

        return carry

    jax.lax.fori_loop(0, chunk, body, 0)


def _k3_body(ct_ref, s_ref, t_ref, r_ref, w_ref, macc_ref, *, chunk, e, n, nand, big):
    @pl.when(pl.program_id(0) == 0)
    def _():
        macc_ref[...] = jnp.full_like(macc_ref, big)

    base = pl.program_id(0) * chunk

    def body(i, carry):
        r = r_ref[0, 0, i]
        valid = (base + i < e) & (r < nand)

        @pl.when(valid)
        def _():
            s = s_ref[0, 0, i]
            t = t_ref[0, 0, i]
            w = w_ref[0, 0, i]
            val = ct_ref[pl.ds(r * n + s, 1), :] * w
            mseg = t * nand + r
            macc_ref[pl.ds(mseg, 1), :] = jnp.minimum(macc_ref[pl.ds(mseg, 1), :], val)

        return carry

    jax.lax.fori_loop(0, chunk, body, 0)


def _k4_body(h_ref, raw_ref, rawb_ref, cnt_ref, cntb_ref, macc_ref, sf_ref,
             wself_ref, wrel_ref, wp_ref, wc_ref, istw_ref, istb_ref,
             lng_ref, lnb_ref, out_ref, *, nrel, nand, dout):
    x = h_ref[...]
    acc = jax.lax.dot_general(x, wself_ref[...], (((1,), (1,)), ((), ())))
    cnt = cnt_ref[...] + cntb_ref[...]
    pos = cnt > 0.0
    denom = jnp.maximum(cnt, 1.0)
    for r in range(nand):
        acc = acc + jnp.where(pos[:, r:r + 1], macc_ref[:, r, :], 0.0)
    m_mat = jax.lax.dot_general(wp_ref[...], wc_ref[...], (((0,), (0,)), ((), ())))
    sitg = jax.nn.sigmoid(
        jax.lax.dot_general(sf_ref[...], istw_ref[...], (((1,), (1,)), ((), ())))
        + istb_ref[...])
    inv_sqrt_d = 1.0 / (float(dout) ** 0.5)
    for r in range(nand, nrel):
        raw_r = raw_ref[:, r - nand, :] + rawb_ref[:, r - nand, :]
        mean_r = jax.lax.dot_general(raw_r, wrel_ref[r],
                                     (((1,), (1,)), ((), ()))) / denom[:, r:r + 1]
        if r in (3, 4):
            th = jax.lax.dot_general(x, wrel_ref[r], (((1,), (1,)), ((), ())))
            attn = jnp.sum(jnp.dot(mean_r, m_mat) * th, axis=-1, keepdims=True)
            g = jax.nn.sigmoid(attn * inv_sqrt_d)
            msg = th * g + mean_r * (1.0 - g)
        elif r == 5:
            msg = mean_r * sitg
        else:
            msg = mean_r
        acc = acc + jnp.where(pos[:, r:r + 1], msg, 0.0)
    o = jnp.maximum(acc, 0.0)
    mu = jnp.mean(o, axis=-1, keepdims=True)
    var = jnp.mean((o - mu) ** 2, axis=-1, keepdims=True)
    out_ref[...] = (o - mu) * jax.lax.rsqrt(var + 1e-5) * lng_ref[...] + lnb_ref[...]


def kernel(h, edge_index, edge_types, edge_weights, situation_features,
           W_self, W_rel, and_W, and_b, if_Wp, if_Wc, ist_W, ist_b, ln_g, ln_b):
    n, din = h.shape
    e = edge_types.shape[0]
    r = W_rel.shape[0]
    dout = W_self.shape[0]
    sit = situation_features.shape[1]
    nand = 3
    nmean = r - nand
    blk = 1000 if n % 1000 == 0 else n
    nblk = n // blk

    f32 = jnp.float32
    i32 = jnp.int32

    ct = pl.pallas_call(
        functools.partial(_k1_body, nand=nand),
        grid=(nblk,),
        in_specs=[
            pl.BlockSpec((blk, din), lambda i: (i, 0)),
            pl.BlockSpec((r, dout, din), lambda i: (0, 0, 0)),
            pl.BlockSpec((dout, dout), lambda i: (0, 0)),
            pl.BlockSpec((1, dout), lambda i: (0, 0)),
        ],
        out_specs=pl.BlockSpec((nand, blk, dout), lambda i: (0, i, 0)),
        out_shape=jax.ShapeDtypeStruct((nand, n, dout), f32),
    )(h, W_rel, and_W, and_b.reshape(1, dout))

    chunk = 5000 if e % 5000 == 0 else e
    nch = -(-e // chunk)
    pad = nch * chunk - e
    src = jnp.pad(edge_index[0].astype(i32), (0, pad)).reshape(nch, 1, chunk)
    tgt = jnp.pad(edge_index[1].astype(i32), (0, pad)).reshape(nch, 1, chunk)
    rel = jnp.pad(edge_types.astype(i32), (0, pad)).reshape(nch, 1, chunk)
    wgt = jnp.pad(edge_weights.astype(f32), (0, pad)).reshape(nch, 1, chunk)

    smem_idx = pl.BlockSpec((1, 1, chunk), lambda i: (i, 0, 0),
                            memory_space=pltpu.SMEM)

    if e >= 4096:
        raw2, cnt2 = _sc_edge_sums(h, edge_index[0].astype(i32),
                                   edge_index[1].astype(i32),
                                   edge_types.astype(i32), n, e)
        raw_a, raw_b = raw2[0], raw2[1]
        cnt_a, cnt_b = cnt2[0], cnt2[1]
    else:
        rawsum, cnt = pl.pallas_call(
            functools.partial(_k2_body, chunk=chunk, e=e, nmean=nmean,
                              rbase=nand),
            grid=(nch,),
            in_specs=[
                pl.BlockSpec((n, din), lambda i: (0, 0)),
                smem_idx, smem_idx, smem_idx,
            ],
            out_specs=[
                pl.BlockSpec((n * nmean, din), lambda i: (0, 0)),
                pl.BlockSpec((n * r // 128, 128), lambda i: (0, 0)),
            ],
            out_shape=[
                jax.ShapeDtypeStruct((n * nmean, din), f32),
                jax.ShapeDtypeStruct((n * r // 128, 128), f32),
            ],
        )(h, src, tgt, rel)
        raw_a = rawsum.reshape(n, nmean, din)
        raw_b = jnp.zeros_like(raw_a)
        cnt_a = cnt.reshape(n, r)
        cnt_b = jnp.zeros_like(cnt_a)

    big = 3.0e38
    macc = pl.pallas_call(
        functools.partial(_k3_body, chunk=chunk, e=e, n=n, nand=nand, big=big),
        grid=(nch,),
        in_specs=[
            pl.BlockSpec((nand * n, dout), lambda i: (0, 0)),
            smem_idx, smem_idx, smem_idx, smem_idx,
        ],
        out_specs=pl.BlockSpec((nand * n, dout), lambda i: (0, 0)),
        out_shape=jax.ShapeDtypeStruct((nand * n, dout), f32),
    )(ct.reshape(nand * n, dout), src, tgt, rel, wgt)

    out = pl.pallas_call(
        functools.partial(_k4_body, nrel=r, nand=nand, dout=dout),
        grid=(nblk,),
        in_specs=[
            pl.BlockSpec((blk, din), lambda i: (i, 0)),
            pl.BlockSpec((blk, nmean, din), lambda i: (i, 0, 0)),
            pl.BlockSpec((blk, nmean, din), lambda i: (i, 0, 0)),
            pl.BlockSpec((blk, r), lambda i: (i, 0)),
            pl.BlockSpec((blk, r), lambda i: (i, 0)),
            pl.BlockSpec((blk, nand, dout), lambda i: (i, 0, 0)),
            pl.BlockSpec((blk, sit), lambda i: (i, 0)),
            pl.BlockSpec((dout, din), lambda i: (0, 0)),
            pl.BlockSpec((r, dout, din), lambda i: (0, 0, 0)),
            pl.BlockSpec((dout, dout), lambda i: (0, 0)),
            pl.BlockSpec((dout, dout), lambda i: (0, 0)),
            pl.BlockSpec((dout, sit), lambda i: (0, 0)),
            pl.BlockSpec((1, dout), lambda i: (0, 0)),
            pl.BlockSpec((1, dout), lambda i: (0, 0)),
            pl.BlockSpec((1, dout), lambda i: (0, 0)),
        ],
        out_specs=pl.BlockSpec((blk, dout), lambda i: (i, 0)),
        out_shape=jax.ShapeDtypeStruct((n, dout), f32),
    )(h, raw_a, raw_b, cnt_a, cnt_b,
      macc.reshape(n, nand, dout), situation_features, W_self, W_rel,
      if_Wp, if_Wc, ist_W, ist_b.reshape(1, dout), ln_g.reshape(1, dout),
      ln_b.reshape(1, dout))
    return out


# speedup vs baseline: 2.1117x; 1.0035x over previous
import functools
import jax
import jax.numpy as jnp
from jax import lax
from jax.experimental import pallas as pl
from jax.experimental.pallas import tpu as pltpu
from jax.experimental.pallas import tpu_sc as plsc


def _sc_edge_body(n, krows, ht_hbm, src_hbm, tgt_hbm, rel_hbm, zeros_hbm,
                  sum_hbm, cnt_hbm, shared, srcv, tgtv, relv, sidx, aux,
                  rows, ones_v, sem):
    c = lax.axis_index("c")
    s = lax.axis_index("s")
    wid = s * 2 + c
    nsum = n * 5
    sum_zone = nsum + 16
    sum_dump = nsum + 8
    nhalf = n * 4
    cnt_dump = nhalf

    pltpu.sync_copy(src_hbm.at[wid], srcv)
    pltpu.sync_copy(tgt_hbm.at[wid], tgtv)
    pltpu.sync_copy(rel_hbm.at[wid], relv)

    def jbody(j, carry):
        def lbody(l, cc):
            t = tgtv[j, pl.ds(l * 16, 16)]
            r = relv[j, pl.ds(l * 16, 16)]
            sidx[j, pl.ds(l * 16, 16)] = jnp.where(
                (r >= 3) & (r < 8), t * 5 + (r - 3), sum_dump)
            return cc
        return lax.fori_loop(0, 8, lbody, carry)
    lax.fori_loop(0, krows, jbody, 0)

    for p in range(8):
        @pl.when(s == 0)
        def _():
            pltpu.sync_copy(zeros_hbm.at[pl.ds(0, sum_zone)],
                            shared.at[pl.ds(0, sum_zone)])
        plsc.subcore_barrier()

        def fill(j, carry):
            def lb(l, cc):
                aux[j, pl.ds(l * 16, 16)] = srcv[j, pl.ds(l * 16, 16)] + p * n
                return cc
            return lax.fori_loop(0, 8, lb, carry)
        lax.fori_loop(0, krows, fill, 0)

        def stream(j, carry):
            pltpu.async_copy(ht_hbm.at[aux.at[j]], rows, sem).wait()
            pltpu.sync_copy(rows, shared.at[sidx.at[j]], add=True)
            return carry
        lax.fori_loop(0, krows, stream, 0)

        plsc.subcore_barrier()

        @pl.when(s == 0)
        def _():
            pltpu.sync_copy(shared.at[pl.ds(0, nsum)], sum_hbm.at[c, p])
        plsc.subcore_barrier()

    def ob(j, carry):
        ones_v[j, :] = jnp.full((16,), 1.0, jnp.float32)
        return carry
    lax.fori_loop(0, 128, ob, 0)

    for ch in range(2):
        @pl.when(s == 0)
        def _():
            pltpu.sync_copy(zeros_hbm.at[pl.ds(0, nhalf + 8)],
                            shared.at[pl.ds(0, nhalf + 8)])
        plsc.subcore_barrier()

        def cfill(j, carry):
            def lb(l, cc):
                t = tgtv[j, pl.ds(l * 16, 16)]
                r = relv[j, pl.ds(l * 16, 16)]
                ci = t * 8 + r - ch * nhalf
                ok = (r < 8) & (ci >= 0) & (ci < nhalf)
                aux[j, pl.ds(l * 16, 16)] = jnp.where(ok, ci, cnt_dump)
                return cc
            return lax.fori_loop(0, 8, lb, carry)
        lax.fori_loop(0, krows, cfill, 0)

        def streamb(j, carry):
            pltpu.sync_copy(ones_v, shared.at[aux.at[j]], add=True)
            return carry
        lax.fori_loop(0, krows, streamb, 0)

        plsc.subcore_barrier()

        @pl.when(s == 0)
        def _():
            pltpu.sync_copy(shared.at[pl.ds(0, nhalf)], cnt_hbm.at[c, ch])
        plsc.subcore_barrier()


def _sc_edge_sums(h, src, tgt, rel, n, e):
    f32, i32 = jnp.float32, jnp.int32
    e2 = -(-e // 4096) * 4096
    slab = e2 // 32
    krows = slab // 128
    pad = e2 - e
    src2 = jnp.pad(src, (0, pad)).reshape(32, krows, 128)
    tgt2 = jnp.pad(tgt, (0, pad)).reshape(32, krows, 128)
    rel2 = jnp.pad(rel, (0, pad), constant_values=1000).reshape(32, krows, 128)
    ht = h.reshape(n, 8, 16).transpose(1, 0, 2).reshape(8 * n, 16)
    zrows = n * 5 + 16
    zeros = jnp.zeros((zrows, 16), f32)

    kern = functools.partial(
        pl.kernel,
        mesh=plsc.VectorSubcoreMesh(core_axis_name="c", subcore_axis_name="s"),
        compiler_params=pltpu.CompilerParams(use_tc_tiling_on_sc=False),
        out_type=[
            jax.ShapeDtypeStruct((2, 8, n * 5, 16), f32),
            jax.ShapeDtypeStruct((2, 2, n * 4, 16), f32),
        ],
        scratch_types=[
            pltpu.VMEM_SHARED((zrows, 16), f32),
            pltpu.VMEM((krows, 128), i32),
            pltpu.VMEM((krows, 128), i32),
            pltpu.VMEM((krows, 128), i32),
            pltpu.VMEM((krows, 128), i32),
            pltpu.VMEM((krows, 128), i32),
            pltpu.VMEM((128, 16), f32),
            pltpu.VMEM((128, 16), f32),
            pltpu.SemaphoreType.DMA,
        ],
    )(functools.partial(_sc_edge_body, n, krows))
    sums, cnt = kern(ht, src2, tgt2, rel2, zeros)
    raw = sums.transpose(0, 2, 1, 3).reshape(2, n, 5, 128)
    cnt2 = cnt[:, :, :, 0].reshape(2, n, 8)
    return raw, cnt2


def _k1_body(h_ref, wrel_ref, andw_ref, andb_ref, ct_ref, *, nand):
    x = h_ref[...]
    for r in range(nand):
        hr = jax.lax.dot_general(x, wrel_ref[r], (((1,), (1,)), ((), ())))
        a = jax.lax.dot_general(hr, andw_ref[...], (((1,), (1,)), ((), ())))
        ct_ref[r] = hr * jax.nn.sigmoid(a + andb_ref[...])


def _k2_body(h_ref, s_ref, t_ref, r_ref, raw_ref, cnt_ref, *, chunk, e, nmean, rbase):
    @pl.when(pl.program_id(0) == 0)
    def _():
        raw_ref[...] = jnp.zeros_like(raw_ref)
        cnt_ref[...] = jnp.zeros_like(cnt_ref)

    base = pl.program_id(0) * chunk

    def body(i, carry):
        s = s_ref[0, 0, i]
        t = t_ref[0, 0, i]
        r = r_ref[0, 0, i]
        valid = base + i < e
        seg = t * 8 + r
        row = seg // 128
        col = seg % 128
        onehot = (jax.lax.broadcasted_iota(jnp.int32, (1, 128), 1) == col)

        @pl.when(valid)
        def _():
            cnt_ref[pl.ds(row, 1), :] += onehot.astype(jnp.float32)

        @pl.when(valid & (r >= rbase))
        def _():
            seg5 = t * nmean + (r - rbase)
            raw_ref[pl.ds(seg5, 1), :] += h_ref[pl.ds(s, 1), :]

        return carry

    jax.lax.fori_loop(0, chunk, body, 0)


def _k3_body(ct_ref, s_ref, t_ref, r_ref, w_ref, macc_ref, *, chunk, e, n, nand, big):
    @pl.when(pl.program_id(0) == 0)
    def _():
        macc_ref[...] = jnp.full_like(macc_ref, big)

    base = pl.program_id(0) * chunk

    def body(i, carry):
        r = r_ref[0, 0, i]
        valid = (base + i < e) & (r < nand)

        @pl.when(valid)
        def _():
            s = s_ref[0, 0, i]
            t = t_ref[0, 0, i]
            w = w_ref[0, 0, i]
            val = ct_ref[pl.ds(r * n + s, 1), :] * w
            mseg = t * nand + r
            macc_ref[pl.ds(mseg, 1), :] = jnp.minimum(macc_ref[pl.ds(mseg, 1), :], val)

        return carry

    jax.lax.fori_loop(0, chunk, body, 0)


def _k4_body(h_ref, raw_ref, rawb_ref, cnt_ref, cntb_ref, macc_ref, sf_ref,
             wself_ref, wrel_ref, wp_ref, wc_ref, istw_ref, istb_ref,
             lng_ref, lnb_ref, out_ref, *, nrel, nand, dout):
    x = h_ref[...]
    acc = jax.lax.dot_general(x, wself_ref[...], (((1,), (1,)), ((), ())))
    cnt = cnt_ref[...] + cntb_ref[...]
    pos = cnt > 0.0
    denom = jnp.maximum(cnt, 1.0)
    for r in range(nand):
        acc = acc + jnp.where(pos[:, r:r + 1], macc_ref[:, r, :], 0.0)
    m_mat = jax.lax.dot_general(wp_ref[...], wc_ref[...], (((0,), (0,)), ((), ())))
    sitg = jax.nn.sigmoid(
        jax.lax.dot_general(sf_ref[...], istw_ref[...], (((1,), (1,)), ((), ())))
        + istb_ref[...])
    inv_sqrt_d = 1.0 / (float(dout) ** 0.5)
    for r in range(nand, nrel):
        raw_r = raw_ref[:, r - nand, :] + rawb_ref[:, r - nand, :]
        mean_r = jax.lax.dot_general(raw_r, wrel_ref[r],
                                     (((1,), (1,)), ((), ()))) / denom[:, r:r + 1]
        if r in (3, 4):
            th = jax.lax.dot_general(x, wrel_ref[r], (((1,), (1,)), ((), ())))
            attn = jnp.sum(jnp.dot(mean_r, m_mat) * th, axis=-1, keepdims=True)
            g = jax.nn.sigmoid(attn * inv_sqrt_d)
            msg = th * g + mean_r * (1.0 - g)
        elif r == 5:
            msg = mean_r * sitg
        else:
            msg = mean_r
        acc = acc + jnp.where(pos[:, r:r + 1], msg, 0.0)
    o = jnp.maximum(acc, 0.0)
    mu = jnp.mean(o, axis=-1, keepdims=True)
    var = jnp.mean((o - mu) ** 2, axis=-1, keepdims=True)
    out_ref[...] = (o - mu) * jax.lax.rsqrt(var + 1e-5) * lng_ref[...] + lnb_ref[...]


def kernel(h, edge_index, edge_types, edge_weights, situation_features,
           W_self, W_rel, and_W, and_b, if_Wp, if_Wc, ist_W, ist_b, ln_g, ln_b):
    n, din = h.shape
    e = edge_types.shape[0]
    r = W_rel.shape[0]
    dout = W_self.shape[0]
    sit = situation_features.shape[1]
    nand = 3
    nmean = r - nand
    blk = 1000 if n % 1000 == 0 else n
    nblk = n // blk

    f32 = jnp.float32
    i32 = jnp.int32

    ct = pl.pallas_call(
        functools.partial(_k1_body, nand=nand),
        grid=(nblk,),
        in_specs=[
            pl.BlockSpec((blk, din), lambda i: (i, 0)),
            pl.BlockSpec((r, dout, din), lambda i: (0, 0, 0)),
            pl.BlockSpec((dout, dout), lambda i: (0, 0)),
            pl.BlockSpec((1, dout), lambda i: (0, 0)),
        ],
        out_specs=pl.BlockSpec((nand, blk, dout), lambda i: (0, i, 0)),
        out_shape=jax.ShapeDtypeStruct((nand, n, dout), f32),
    )(h, W_rel, and_W, and_b.reshape(1, dout))

    chunk = 10000 if e % 10000 == 0 else e
    nch = -(-e // chunk)
    pad = nch * chunk - e
    src = jnp.pad(edge_index[0].astype(i32), (0, pad)).reshape(nch, 1, chunk)
    tgt = jnp.pad(edge_index[1].astype(i32), (0, pad)).reshape(nch, 1, chunk)
    rel = jnp.pad(edge_types.astype(i32), (0, pad)).reshape(nch, 1, chunk)
    wgt = jnp.pad(edge_weights.astype(f32), (0, pad)).reshape(nch, 1, chunk)

    smem_idx = pl.BlockSpec((1, 1, chunk), lambda i: (i, 0, 0),
                            memory_space=pltpu.SMEM)

    if e >= 4096:
        raw2, cnt2 = _sc_edge_sums(h, edge_index[0].astype(i32),
                                   edge_index[1].astype(i32),
                                   edge_types.astype(i32), n, e)
        raw_a, raw_b = raw2[0], raw2[1]
        cnt_a, cnt_b = cnt2[0], cnt2[1]
    else:
        rawsum, cnt = pl.pallas_call(
            functools.partial(_k2_body, chunk=chunk, e=e, nmean=nmean,
                              rbase=nand),
            grid=(nch,),
            in_specs=[
                pl.BlockSpec((n, din), lambda i: (0, 0)),
                smem_idx, smem_idx, smem_idx,
            ],
            out_specs=[
                pl.BlockSpec((n * nmean, din), lambda i: (0, 0)),
                pl.BlockSpec((n * r // 128, 128), lambda i: (0, 0)),
            ],
            out_shape=[
                jax.ShapeDtypeStruct((n * nmean, din), f32),
                jax.ShapeDtypeStruct((n * r // 128, 128), f32),
            ],
        )(h, src, tgt, rel)
        raw_a = rawsum.reshape(n, nmean, din)
        raw_b = jnp.zeros_like(raw_a)
        cnt_a = cnt.reshape(n, r)
        cnt_b = jnp.zeros_like(cnt_a)

    big = 3.0e38
    macc = pl.pallas_call(
        functools.partial(_k3_body, chunk=chunk, e=e, n=n, nand=nand, big=big),
        grid=(nch,),
        in_specs=[
            pl.BlockSpec((nand * n, dout), lambda i: (0, 0)),
            smem_idx, smem_idx, smem_idx, smem_idx,
        ],
        out_specs=pl.BlockSpec((nand * n, dout), lambda i: (0, 0)),
        out_shape=jax.ShapeDtypeStruct((nand * n, dout), f32),
    )(ct.reshape(nand * n, dout), src, tgt, rel, wgt)

    out = pl.pallas_call(
        functools.partial(_k4_body, nrel=r, nand=nand, dout=dout),
        grid=(nblk,),
        in_specs=[
            pl.BlockSpec((blk, din), lambda i: (i, 0)),
            pl.BlockSpec((blk, nmean, din), lambda i: (i, 0, 0)),
            pl.BlockSpec((blk, nmean, din), lambda i: (i, 0, 0)),
            pl.BlockSpec((blk, r), lambda i: (i, 0)),
            pl.BlockSpec((blk, r), lambda i: (i, 0)),
            pl.BlockSpec((blk, nand, dout), lambda i: (i, 0, 0)),
            pl.BlockSpec((blk, sit), lambda i: (i, 0)),
            pl.BlockSpec((dout, din), lambda i: (0, 0)),
            pl.BlockSpec((r, dout, din), lambda i: (0, 0, 0)),
            pl.BlockSpec((dout, dout), lambda i: (0, 0)),
            pl.BlockSpec((dout, dout), lambda i: (0, 0)),
            pl.BlockSpec((dout, sit), lambda i: (0, 0)),
            pl.BlockSpec((1, dout), lambda i: (0, 0)),
            pl.BlockSpec((1, dout), lambda i: (0, 0)),
            pl.BlockSpec((1, dout), lambda i: (0, 0)),
        ],
        out_specs=pl.BlockSpec((blk, dout), lambda i: (i, 0)),
        out_shape=jax.ShapeDtypeStruct((n, dout), f32),
    )(h, raw_a, raw_b, cnt_a, cnt_b,
      macc.reshape(n, nand, dout), situation_features, W_self, W_rel,
      if_Wp, if_Wc, ist_W, ist_b.reshape(1, dout), ln_g.reshape(1, dout),
      ln_b.reshape(1, dout))
    return out
